# Initial kernel scaffold; baseline (speedup 1.0000x reference)
import jax
import jax.numpy as jnp
from jax.experimental import pallas as pl


def kernel(inputs, table):
    raise NotImplementedError("write your pallas kernel here")


# speedup vs baseline: 177.4225x; 177.4225x over previous
import functools

import jax
import jax.numpy as jnp
from jax import lax
from jax.experimental import pallas as pl
from jax.experimental.pallas import tpu as pltpu
from jax.experimental.pallas import tpu_sc as plsc

_NUM_CORES = 2
_NUM_SUBCORES = 16
_NW = _NUM_CORES * _NUM_SUBCORES
_L = 16
_TABLE_PAD = 128


def _lookup_call(flat_idx, table_padded, n):
    per_w = n // _NW
    iters = per_w // _L
    mesh = plsc.VectorSubcoreMesh(core_axis_name="c", subcore_axis_name="s")

    @functools.partial(
        pl.kernel,
        mesh=mesh,
        out_type=jax.ShapeDtypeStruct((n,), jnp.float32),
        scratch_types=[
            pltpu.VMEM((per_w,), jnp.int32),
            pltpu.VMEM((per_w,), jnp.float32),
            pltpu.VMEM((_TABLE_PAD,), jnp.float32),
        ],
        compiler_params=pltpu.CompilerParams(needs_layout_passes=False),
    )
    def _lookup(idx_hbm, table_hbm, out_hbm, idx_v, out_v, table_v):
        wid = lax.axis_index("s") * _NUM_CORES + lax.axis_index("c")
        base = wid * per_w
        pltpu.sync_copy(table_hbm, table_v)
        pltpu.sync_copy(idx_hbm.at[pl.ds(base, per_w)], idx_v)

        def body(i, carry):
            off = i * _L
            v = idx_v[pl.ds(off, _L)]
            out_v[pl.ds(off, _L)] = plsc.load_gather(table_v, [v])
            return carry

        lax.fori_loop(0, iters, body, 0)
        pltpu.sync_copy(out_v, out_hbm.at[pl.ds(base, per_w)])

    return _lookup(flat_idx, table_padded)


def kernel(inputs, table):
    b, f = inputs.shape
    n = b * f
    flat_idx = inputs.reshape(n)
    table_padded = jnp.zeros((_TABLE_PAD,), jnp.float32).at[: table.shape[0]].set(table)
    out = _lookup_call(flat_idx, table_padded, n)
    return out.reshape(b, f)


# speedup vs baseline: 212.3784x; 1.1970x over previous
import functools

import jax
import jax.numpy as jnp
from jax import lax
from jax.experimental import pallas as pl
from jax.experimental.pallas import tpu as pltpu
from jax.experimental.pallas import tpu_sc as plsc

_NUM_CORES = 2
_NUM_SUBCORES = 16
_NW = _NUM_CORES * _NUM_SUBCORES
_L = 16
_TABLE_PAD = 128


def _lookup_call(flat_idx, table_padded, n):
    per_w = n // _NW
    iters = per_w // _L
    mesh = plsc.VectorSubcoreMesh(core_axis_name="c", subcore_axis_name="s")

    @functools.partial(
        pl.kernel,
        mesh=mesh,
        out_type=jax.ShapeDtypeStruct((n,), jnp.float32),
        scratch_types=[
            pltpu.VMEM((per_w,), jnp.int32),
            pltpu.VMEM((per_w,), jnp.float32),
            pltpu.VMEM((_TABLE_PAD,), jnp.float32),
        ],
        compiler_params=pltpu.CompilerParams(needs_layout_passes=False),
    )
    def _lookup(idx_hbm, table_hbm, out_hbm, idx_v, out_v, table_v):
        wid = lax.axis_index("s") * _NUM_CORES + lax.axis_index("c")
        base = wid * per_w
        pltpu.sync_copy(table_hbm, table_v)
        pltpu.sync_copy(idx_hbm.at[pl.ds(base, per_w)], idx_v)

        @plsc.parallel_loop(0, per_w, _L, unroll=8)
        def _body(off):
            v = idx_v[pl.ds(off, _L)]
            out_v[pl.ds(off, _L)] = plsc.load_gather(table_v, [v])
        pltpu.sync_copy(out_v, out_hbm.at[pl.ds(base, per_w)])

    return _lookup(flat_idx, table_padded)


def kernel(inputs, table):
    b, f = inputs.shape
    n = b * f
    flat_idx = inputs.reshape(n)
    table_padded = jnp.zeros((_TABLE_PAD,), jnp.float32).at[: table.shape[0]].set(table)
    out = _lookup_call(flat_idx, table_padded, n)
    return out.reshape(b, f)


# speedup vs baseline: 347.0362x; 1.6340x over previous
import functools

import jax
import jax.numpy as jnp
from jax import lax
from jax.experimental import pallas as pl
from jax.experimental.pallas import tpu as pltpu
from jax.experimental.pallas import tpu_sc as plsc

_NUM_CORES = 2
_NUM_SUBCORES = 16
_NW = _NUM_CORES * _NUM_SUBCORES
_L = 16
_TABLE_PAD = 128
_CHUNK_ROWS = 256


def _row_offsets(width):
    offs = list(range(0, width - _L + 1, _L))
    if offs[-1] != width - _L:
        offs.append(width - _L)
    return offs


def _lookup_call(idx2d, table_padded):
    n_rows, width = idx2d.shape
    rows_per_w = n_rows // _NW
    n_chunks = rows_per_w // _CHUNK_ROWS
    offs = _row_offsets(width)
    mesh = plsc.VectorSubcoreMesh(core_axis_name="c", subcore_axis_name="s")

    @functools.partial(
        pl.kernel,
        mesh=mesh,
        out_type=jax.ShapeDtypeStruct((n_rows, width), jnp.float32),
        scratch_types=[
            pltpu.VMEM((_CHUNK_ROWS, width), jnp.int32),
            pltpu.VMEM((_CHUNK_ROWS, width), jnp.float32),
            pltpu.VMEM((_TABLE_PAD,), jnp.float32),
        ],
        compiler_params=pltpu.CompilerParams(needs_layout_passes=False),
    )
    def _lookup(idx_hbm, table_hbm, out_hbm, idx_v, out_v, table_v):
        wid = lax.axis_index("s") * _NUM_CORES + lax.axis_index("c")
        base = wid * rows_per_w
        pltpu.sync_copy(table_hbm, table_v)

        def chunk_body(c, carry):
            row0 = base + c * _CHUNK_ROWS
            pltpu.sync_copy(idx_hbm.at[pl.ds(row0, _CHUNK_ROWS), :], idx_v)

            @plsc.parallel_loop(0, _CHUNK_ROWS, 1, unroll=4)
            def _body(r):
                for off in offs:
                    v = idx_v[r, pl.ds(off, _L)]
                    out_v[r, pl.ds(off, _L)] = plsc.load_gather(table_v, [v])

            pltpu.sync_copy(out_v, out_hbm.at[pl.ds(row0, _CHUNK_ROWS), :])
            return carry

        lax.fori_loop(0, n_chunks, chunk_body, 0)

    return _lookup(idx2d, table_padded)


def kernel(inputs, table):
    table_padded = jnp.zeros((_TABLE_PAD,), jnp.float32).at[: table.shape[0]].set(table)
    return _lookup_call(inputs, table_padded)


# speedup vs baseline: 523.3749x; 1.5081x over previous
import functools

import jax
import jax.numpy as jnp
from jax import lax
from jax.experimental import pallas as pl
from jax.experimental.pallas import tpu as pltpu
from jax.experimental.pallas import tpu_sc as plsc

_NUM_CORES = 2
_NUM_SUBCORES = 16
_NW = _NUM_CORES * _NUM_SUBCORES
_L = 16
_TABLE_PAD = 128


def _lookup_call(idx_t, table_padded):
    n_rows, n_cols = idx_t.shape
    cols_per_w = n_cols // _NW
    vecs_per_row = cols_per_w // _L
    mesh = plsc.VectorSubcoreMesh(core_axis_name="c", subcore_axis_name="s")

    @functools.partial(
        pl.kernel,
        mesh=mesh,
        out_type=jax.ShapeDtypeStruct((n_rows, n_cols), jnp.float32),
        scratch_types=[
            pltpu.VMEM((n_rows, cols_per_w), jnp.int32),
            pltpu.VMEM((n_rows, cols_per_w), jnp.float32),
            pltpu.VMEM((_TABLE_PAD,), jnp.float32),
        ],
        compiler_params=pltpu.CompilerParams(needs_layout_passes=False),
    )
    def _lookup(idx_hbm, table_hbm, out_hbm, idx_v, out_v, table_v):
        wid = lax.axis_index("s") * _NUM_CORES + lax.axis_index("c")
        base = wid * cols_per_w
        pltpu.sync_copy(table_hbm, table_v)
        pltpu.sync_copy(idx_hbm.at[:, pl.ds(base, cols_per_w)], idx_v)

        @plsc.parallel_loop(0, n_rows, 1, unroll=2)
        def _body(r):
            for j in range(vecs_per_row):
                v = idx_v[r, pl.ds(j * _L, _L)]
                out_v[r, pl.ds(j * _L, _L)] = plsc.load_gather(table_v, [v])

        pltpu.sync_copy(out_v, out_hbm.at[:, pl.ds(base, cols_per_w)])

    return _lookup(idx_t, table_padded)


def kernel(inputs, table):
    table_padded = jnp.zeros((_TABLE_PAD,), jnp.float32).at[: table.shape[0]].set(table)
    out_t = _lookup_call(inputs.T, table_padded)
    return out_t.T


# speedup vs baseline: 541.5264x; 1.0347x over previous
import functools

import jax
import jax.numpy as jnp
from jax import lax
from jax.experimental import pallas as pl
from jax.experimental.pallas import tpu as pltpu
from jax.experimental.pallas import tpu_sc as plsc

_NUM_CORES = 2
_NUM_SUBCORES = 16
_NW = _NUM_CORES * _NUM_SUBCORES
_L = 16
_NCHUNK = 4


def _lookup_call(idx_t, table):
    n_rows, n_cols = idx_t.shape
    table_n = table.shape[0]
    cols_per_w = n_cols // _NW
    chunk = cols_per_w // _NCHUNK
    vecs = chunk // _L
    mesh = plsc.VectorSubcoreMesh(core_axis_name="c", subcore_axis_name="s")

    @functools.partial(
        pl.kernel,
        mesh=mesh,
        out_type=jax.ShapeDtypeStruct((n_rows, n_cols), jnp.float32),
        scratch_types=[
            [pltpu.VMEM((n_rows, chunk), jnp.int32) for _ in range(_NCHUNK)],
            [pltpu.VMEM((n_rows, chunk), jnp.float32) for _ in range(_NCHUNK)],
            pltpu.VMEM((table_n,), jnp.float32),
            [pltpu.SemaphoreType.DMA for _ in range(_NCHUNK)],
            [pltpu.SemaphoreType.DMA for _ in range(_NCHUNK)],
        ],
        compiler_params=pltpu.CompilerParams(
            needs_layout_passes=False, skip_device_barrier=True
        ),
    )
    def _lookup(idx_hbm, table_hbm, out_hbm, idx_bufs, out_bufs, table_v,
                in_sems, out_sems):
        wid = lax.axis_index("s") * _NUM_CORES + lax.axis_index("c")
        base = wid * cols_per_w
        pltpu.sync_copy(table_hbm, table_v)

        in_handles = [
            pltpu.async_copy(
                idx_hbm.at[:, pl.ds(base + h * chunk, chunk)],
                idx_bufs[h],
                in_sems[h],
            )
            for h in range(_NCHUNK)
        ]
        out_handles = []
        for h in range(_NCHUNK):
            in_handles[h].wait()
            iv, ov = idx_bufs[h], out_bufs[h]

            @plsc.parallel_loop(0, n_rows, 1, unroll=2)
            def _body(r, iv=iv, ov=ov):
                for j in range(vecs):
                    v = iv[r, pl.ds(j * _L, _L)]
                    ov[r, pl.ds(j * _L, _L)] = plsc.load_gather(table_v, [v])

            out_handles.append(
                pltpu.async_copy(
                    out_bufs[h],
                    out_hbm.at[:, pl.ds(base + h * chunk, chunk)],
                    out_sems[h],
                )
            )
        for hdl in out_handles:
            hdl.wait()

    return _lookup(idx_t, table)


def kernel(inputs, table):
    out_t = _lookup_call(inputs.T, table)
    return out_t.T


# speedup vs baseline: 583.7511x; 1.0780x over previous
import functools

import jax
import jax.numpy as jnp
from jax import lax
from jax.experimental import pallas as pl
from jax.experimental.pallas import tpu as pltpu
from jax.experimental.pallas import tpu_sc as plsc

_NUM_CORES = 2
_NUM_SUBCORES = 16
_NW = _NUM_CORES * _NUM_SUBCORES
_L = 16
_CHUNKS = (128, 128, 128, 128)


def _lookup_call(idx_t, table):
    n_rows, n_cols = idx_t.shape
    table_n = table.shape[0]
    cols_per_w = n_cols // _NW
    assert sum(_CHUNKS) == cols_per_w
    starts = [sum(_CHUNKS[:h]) for h in range(len(_CHUNKS))]
    nchunk = len(_CHUNKS)
    mesh = plsc.VectorSubcoreMesh(core_axis_name="c", subcore_axis_name="s")

    @functools.partial(
        pl.kernel,
        mesh=mesh,
        out_type=jax.ShapeDtypeStruct((n_rows, n_cols), jnp.float32),
        scratch_types=[
            [pltpu.VMEM((n_rows, c), jnp.int32) for c in _CHUNKS],
            [pltpu.VMEM((n_rows, c), jnp.float32) for c in _CHUNKS],
            pltpu.VMEM((table_n,), jnp.float32),
            [pltpu.SemaphoreType.DMA for _ in _CHUNKS],
            [pltpu.SemaphoreType.DMA for _ in _CHUNKS],
        ],
        compiler_params=pltpu.CompilerParams(
            needs_layout_passes=False, skip_device_barrier=True
        ),
    )
    def _lookup(idx_hbm, table_hbm, out_hbm, idx_bufs, out_bufs, table_v,
                in_sems, out_sems):
        wid = lax.axis_index("s") * _NUM_CORES + lax.axis_index("c")
        base = wid * cols_per_w

        def fire_in(h):
            return pltpu.async_copy(
                idx_hbm.at[:, pl.ds(base + starts[h], _CHUNKS[h])],
                idx_bufs[h],
                in_sems[h],
            )

        in_handles = {0: fire_in(0)}
        pltpu.sync_copy(table_hbm, table_v)

        out_handles = []
        for h in range(nchunk):
            if h + 1 < nchunk:
                in_handles[h + 1] = fire_in(h + 1)
            in_handles[h].wait()
            iv, ov = idx_bufs[h], out_bufs[h]
            vecs = _CHUNKS[h] // _L

            @plsc.parallel_loop(0, n_rows, 1, unroll=2)
            def _body(r, iv=iv, ov=ov, vecs=vecs):
                for j in range(vecs):
                    v = iv[r, pl.ds(j * _L, _L)]
                    ov[r, pl.ds(j * _L, _L)] = plsc.load_gather(table_v, [v])

            out_handles.append(
                pltpu.async_copy(
                    out_bufs[h],
                    out_hbm.at[:, pl.ds(base + starts[h], _CHUNKS[h])],
                    out_sems[h],
                )
            )
        for hdl in out_handles:
            hdl.wait()

    return _lookup(idx_t, table)


def kernel(inputs, table):
    out_t = _lookup_call(inputs.T, table)
    return out_t.T
